# Initial kernel scaffold; baseline (speedup 1.0000x reference)
import jax
import jax.numpy as jnp
from jax.experimental import pallas as pl


def kernel(edge_index, edge_type, h, r, t, emb, comp1, bases1, root1, bias1, comp2, bases2, root2, bias2, rel):
    raise NotImplementedError("write your pallas kernel here")


# speedup vs baseline: 3.5089x; 3.5089x over previous
import functools

import jax
import jax.numpy as jnp
from jax import lax
from jax.experimental import pallas as pl
from jax.experimental.pallas import tpu as pltpu
from jax.experimental.pallas import tpu_sc as plsc

N_NODES = 10000
N_REL = 8
N_BASES = 8
D = 256
N_EDGES = 160000
N_TRIPLES = 8192

NC = 2
NS = 16
NW = NC * NS

KC = 80
NE_PAD = 161280
EPW = NE_PAD // NW
EPT = NE_PAD // NS
NCW = EPW // KC
NCT = EPT // KC
CNT_PAD = 81920
ZSLOT = CNT_PAD - 1
NPLANE = N_REL + 1
DUMMY = N_NODES
SPLIT = 5000
P0_ROWS = 5120
P1_ROWS = 5376
TPT = N_TRIPLES // NW
KT = 64


def _sc_mesh():
    return plsc.VectorSubcoreMesh(
        core_axis_name="c", subcore_axis_name="s", num_cores=NC, num_subcores=NS
    )


def _counts(dst, typ):
    @functools.partial(
        pl.kernel,
        out_type=(
            jax.ShapeDtypeStruct((CNT_PAD,), jnp.float32),
            jax.ShapeDtypeStruct((CNT_PAD,), jnp.float32),
        ),
        mesh=_sc_mesh(),
        scratch_types=[
            pltpu.VMEM((EPW,), jnp.int32),
            pltpu.VMEM((EPW,), jnp.int32),
            pltpu.VMEM((KC,), jnp.int32),
            pltpu.VMEM((KC,), jnp.float32),
            pltpu.VMEM((640,), jnp.float32),
            pltpu.VMEM_SHARED((CNT_PAD,), jnp.float32),
        ],
    )
    def body(dst_hbm, typ_hbm, out0, out1, b_dst, b_typ, b_ci, b_one, b_z,
             sh_cnt):
        c = lax.axis_index("c")
        s = lax.axis_index("s")
        zero16 = jnp.zeros((16,), jnp.float32)
        one16 = jnp.ones((16,), jnp.float32)

        def fz(i, _):
            b_z[pl.ds(i * 16, 16)] = zero16
            return 0

        lax.fori_loop(0, 40, fz, 0)

        def fo(i, _):
            b_one[pl.ds(i * 16, 16)] = one16
            return 0

        lax.fori_loop(0, KC // 16, fo, 0)

        def fzs(t, _):
            pltpu.sync_copy(b_z, sh_cnt.at[pl.ds(s * 5120 + t * 640, 640)])
            return 0

        lax.fori_loop(0, 8, fzs, 0)
        plsc.subcore_barrier()

        wid = c * NS + s
        base = wid * EPW
        pltpu.sync_copy(dst_hbm.at[pl.ds(base, EPW)], b_dst)
        pltpu.sync_copy(typ_hbm.at[pl.ds(base, EPW)], b_typ)

        def fidx(i, _):
            sl = pl.ds(i * 16, 16)
            b_dst[sl] = b_dst[sl] * N_REL + b_typ[sl]
            return 0

        lax.fori_loop(0, EPW // 16, fidx, 0)

        def fadd(k, _):
            k0 = k * KC

            def fc(j, _2):
                b_ci[pl.ds(j * 16, 16)] = b_dst[pl.ds(k0 + j * 16, 16)]
                return 0

            lax.fori_loop(0, KC // 16, fc, 0)
            pltpu.sync_copy(b_one, sh_cnt.at[b_ci], add=True)
            return 0

        lax.fori_loop(0, NCW, fadd, 0)
        plsc.subcore_barrier()

        def fout(t, _):
            sl = pl.ds(s * 5120 + t * 640, 640)
            pltpu.sync_copy(sh_cnt.at[sl], b_z)

            @pl.when(c == 0)
            def _():
                pltpu.sync_copy(b_z, out0.at[sl])

            @pl.when(c == 1)
            def _():
                pltpu.sync_copy(b_z, out1.at[sl])

            return 0

        lax.fori_loop(0, 8, fout, 0)

    return body(dst, typ)


def _winv_body(c0_ref, c1_ref, o_ref):
    cnt = c0_ref[...] + c1_ref[...]
    o_ref[...] = jnp.where(cnt > 0.5, 1.0 / cnt, 0.0)


def _winv(c0, c1, interpret=False):
    return pl.pallas_call(
        _winv_body,
        in_specs=[
            pl.BlockSpec(memory_space=pltpu.VMEM),
            pl.BlockSpec(memory_space=pltpu.VMEM),
        ],
        out_specs=pl.BlockSpec(memory_space=pltpu.VMEM),
        out_shape=jax.ShapeDtypeStruct((640, 128), jnp.float32),
        interpret=interpret,
    )(c0, c1)


def _wcat_body(comp_ref, bases_ref, root_ref, o_ref):
    o_ref[:, 0:D] = root_ref[...]
    for rr in range(N_REL):
        acc = comp_ref[rr, 0] * bases_ref[0]
        for b in range(1, N_BASES):
            acc = acc + comp_ref[rr, b] * bases_ref[b]
        o_ref[:, pl.ds(D * (rr + 1), D)] = acc


def _wcat(comp, bases, root, interpret=False):
    return pl.pallas_call(
        _wcat_body,
        in_specs=[
            pl.BlockSpec(memory_space=pltpu.SMEM),
            pl.BlockSpec(memory_space=pltpu.VMEM),
            pl.BlockSpec(memory_space=pltpu.VMEM),
        ],
        out_specs=pl.BlockSpec(memory_space=pltpu.VMEM),
        out_shape=jax.ShapeDtypeStruct((D, NPLANE * D), jnp.float32),
        interpret=interpret,
    )(comp, bases, root)


_BR = 1000


def _mm1_body(x_ref, w_ref, b_ref, o_ref):
    o_ref[...] = (
        jnp.dot(x_ref[...], w_ref[...], preferred_element_type=jnp.float32)
        + b_ref[...]
    )


def _mm1(x, wcat, biascat, interpret=False):
    return pl.pallas_call(
        _mm1_body,
        grid=(N_NODES // _BR,),
        in_specs=[
            pl.BlockSpec((_BR, D), lambda i: (i, 0)),
            pl.BlockSpec((D, NPLANE * D), lambda i: (0, 0)),
            pl.BlockSpec((1, NPLANE * D), lambda i: (0, 0)),
        ],
        out_specs=pl.BlockSpec((_BR, NPLANE * D), lambda i: (i, 0)),
        out_shape=jax.ShapeDtypeStruct((N_NODES, NPLANE * D), jnp.float32),
        interpret=interpret,
    )(x, wcat, biascat)


def _agg_specs():
    return [
        pl.BlockSpec((_BR, D), lambda i: (jnp.minimum(i, 4), 0)),
        pl.BlockSpec((_BR, D), lambda i: (jnp.maximum(i - 5, 0), 0)),
    ]


def _mm2_body(ys_ref, a0_ref, a1_ref, w_ref, b_ref, o_ref):
    agg = jnp.where(pl.program_id(0) < 5, a0_ref[...], a1_ref[...])
    x = jnp.maximum(ys_ref[...] + agg, 0.0)
    o_ref[...] = (
        jnp.dot(x, w_ref[...], preferred_element_type=jnp.float32) + b_ref[...]
    )


def _mm2(y_prev, agg0, agg1, wcat, biascat, interpret=False):
    return pl.pallas_call(
        _mm2_body,
        grid=(N_NODES // _BR,),
        in_specs=[
            pl.BlockSpec((_BR, D), lambda i: (i, 0)),
        ] + _agg_specs() + [
            pl.BlockSpec((D, NPLANE * D), lambda i: (0, 0)),
            pl.BlockSpec((1, NPLANE * D), lambda i: (0, 0)),
        ],
        out_specs=pl.BlockSpec((_BR, NPLANE * D), lambda i: (i, 0)),
        out_shape=jax.ShapeDtypeStruct((N_NODES, NPLANE * D), jnp.float32),
        interpret=interpret,
    )(y_prev, agg0, agg1, wcat, biascat)


def _finalx_body(ys_ref, a0_ref, a1_ref, o_ref):
    agg = jnp.where(pl.program_id(0) < 5, a0_ref[...], a1_ref[...])
    o_ref[...] = jnp.maximum(ys_ref[...] + agg, 0.0)


def _finalx(y_prev, agg0, agg1, interpret=False):
    return pl.pallas_call(
        _finalx_body,
        grid=(N_NODES // _BR,),
        in_specs=[
            pl.BlockSpec((_BR, D), lambda i: (i, 0)),
        ] + _agg_specs(),
        out_specs=pl.BlockSpec((_BR, D), lambda i: (i, 0)),
        out_shape=jax.ShapeDtypeStruct((N_NODES, D), jnp.float32),
        interpret=interpret,
    )(y_prev, agg0, agg1)


DH = 128


def _aggregate(yflat2, src, dst, typ, wtab):
    @functools.partial(
        pl.kernel,
        out_type=(
            jax.ShapeDtypeStruct((P0_ROWS, DH), jnp.float32),
            jax.ShapeDtypeStruct((P0_ROWS, DH), jnp.float32),
            jax.ShapeDtypeStruct((P1_ROWS, DH), jnp.float32),
            jax.ShapeDtypeStruct((P1_ROWS, DH), jnp.float32),
        ),
        mesh=_sc_mesh(),
        scratch_types=[
            pltpu.VMEM((EPT,), jnp.int32),
            pltpu.VMEM((EPT,), jnp.int32),
            pltpu.VMEM((EPT,), jnp.int32),
            pltpu.VMEM((KC,), jnp.int32),
            pltpu.VMEM((KC,), jnp.int32),
            pltpu.VMEM((KC, DH), jnp.float32),
            pltpu.VMEM((KC + 16,), jnp.float32),
            pltpu.VMEM_SHARED((P1_ROWS, DH), jnp.float32),
            pltpu.SemaphoreType.DMA,
        ],
    )
    def body(yflat_hbm, src_hbm, dst_hbm, typ_hbm, wtab_hbm,
             out0a, out0b, out1a, out1b,
             b_g, b_d, b_w, b_ci, b_di, b_rows, b_wc, sh_plane, sem):
        c = lax.axis_index("c")
        s = lax.axis_index("s")
        zero16 = jnp.zeros((16,), jnp.float32)
        lo = c * SPLIT
        nloc = jnp.where(c == 0, P0_ROWS, P1_ROWS)

        base = s * EPT
        pltpu.sync_copy(src_hbm.at[pl.ds(base, EPT)], b_g)
        pltpu.sync_copy(dst_hbm.at[pl.ds(base, EPT)], b_d)
        pltpu.sync_copy(typ_hbm.at[pl.ds(base, EPT)], b_w)

        def ftr(i, _):
            sl = pl.ds(i * 16, 16)
            tv = b_w[sl]
            dv = b_d[sl]
            b_g[sl] = (b_g[sl] * NPLANE + (tv + 1)) * 2
            dloc = dv - lo
            inr = (dloc >= 0) & (dloc < nloc)
            b_w[sl] = jnp.where(inr, dv * N_REL + tv, ZSLOT)
            b_d[sl] = jnp.clip(dloc, 0, nloc - 1)
            return 0

        lax.fori_loop(0, EPT // 16, ftr, 0)

        for f in (0, 1):
            def fzr(rr, _):
                for g in range(DH // 16):
                    b_rows[rr, pl.ds(g * 16, 16)] = zero16
                return 0

            lax.fori_loop(0, KC, fzr, 0)

            def fzp(t, _):
                pltpu.sync_copy(
                    b_rows, sh_plane.at[pl.ds(s * 336 + t * KC, KC)])
                return 0

            lax.fori_loop(0, 4, fzp, 0)
            pltpu.sync_copy(b_rows.at[pl.ds(0, 16)],
                            sh_plane.at[pl.ds(s * 336 + 320, 16)])
            plsc.subcore_barrier()

            def fchunk(k, _):
                k0 = k * KC

                def fc(j, _2):
                    slj = pl.ds(j * 16, 16)
                    slk = pl.ds(k0 + j * 16, 16)
                    b_ci[slj] = b_g[slk] + f
                    b_di[slj] = b_d[slk]
                    return 0

                lax.fori_loop(0, KC // 16, fc, 0)
                cp1 = pltpu.async_copy(yflat_hbm.at[b_ci], b_rows, sem)
                cp2 = pltpu.async_copy(
                    wtab_hbm.at[b_w.at[pl.ds(k0, KC)]], b_wc.at[pl.ds(0, KC)], sem)
                cp1.wait()
                cp2.wait()

                def fscale(rr, _2):
                    wv = b_wc[pl.ds(rr, 16)][0]
                    for g in range(DH // 16):
                        sl = pl.ds(g * 16, 16)
                        b_rows[rr, sl] = b_rows[rr, sl] * wv
                    return 0

                lax.fori_loop(0, KC, fscale, 0)
                pltpu.sync_copy(b_rows, sh_plane.at[b_di], add=True)
                return 0

            lax.fori_loop(0, NCT, fchunk, 0)
            plsc.subcore_barrier()

            outc0 = out0a if f == 0 else out0b
            outc1 = out1a if f == 0 else out1b

            @pl.when(c == 0)
            def _():
                def fo0(t, _):
                    sl = pl.ds(s * 320 + t * KC, KC)
                    pltpu.sync_copy(sh_plane.at[sl], b_rows)
                    pltpu.sync_copy(b_rows, outc0.at[sl])
                    return 0

                lax.fori_loop(0, 4, fo0, 0)

            @pl.when(c == 1)
            def _():
                def fo1(t, _):
                    sl = pl.ds(s * 336 + t * KC, KC)
                    pltpu.sync_copy(sh_plane.at[sl], b_rows)
                    pltpu.sync_copy(b_rows, outc1.at[sl])
                    return 0

                lax.fori_loop(0, 4, fo1, 0)
                sl16 = pl.ds(s * 336 + 320, 16)
                pltpu.sync_copy(sh_plane.at[sl16], b_rows.at[pl.ds(0, 16)])
                pltpu.sync_copy(b_rows.at[pl.ds(0, 16)], outc1.at[sl16])

            plsc.subcore_barrier()

    return body(yflat2, src, dst, typ, wtab)


def _hsum(v):
    tot = v[0]
    for j in range(1, 16):
        tot = tot + v[j]
    return tot


def _decoder(x, relt, h, r, t):
    @functools.partial(
        pl.kernel,
        out_type=jax.ShapeDtypeStruct((N_TRIPLES,), jnp.float32),
        mesh=_sc_mesh(),
        scratch_types=[
            pltpu.VMEM((TPT,), jnp.int32),
            pltpu.VMEM((TPT,), jnp.int32),
            pltpu.VMEM((TPT,), jnp.int32),
            pltpu.VMEM((KT,), jnp.int32),
            pltpu.VMEM((KT,), jnp.int32),
            pltpu.VMEM((KT,), jnp.int32),
            pltpu.VMEM((KT, D), jnp.float32),
            pltpu.VMEM((KT, D), jnp.float32),
            pltpu.VMEM((KT, D), jnp.float32),
            pltpu.VMEM((TPT,), jnp.float32),
            pltpu.SemaphoreType.DMA,
        ],
    )
    def body(x_hbm, rel_hbm, h_hbm, r_hbm, t_hbm, out_hbm,
             b_h, b_r, b_t, b_hi, b_ri, b_ti, b_xh, b_xt, b_rr,
             b_res, sem):
        c = lax.axis_index("c")
        s = lax.axis_index("s")
        wid = c * NS + s
        base = wid * TPT
        pltpu.sync_copy(h_hbm.at[pl.ds(base, TPT)], b_h)
        pltpu.sync_copy(r_hbm.at[pl.ds(base, TPT)], b_r)
        pltpu.sync_copy(t_hbm.at[pl.ds(base, TPT)], b_t)
        iota16 = lax.iota(jnp.int32, 16)

        for k in range(TPT // KT):
            k0 = k * KT
            for g in range(KT // 16):
                sl = pl.ds(g * 16, 16)
                b_hi[sl] = b_h[pl.ds(k0 + g * 16, 16)]
                b_ri[sl] = b_r[pl.ds(k0 + g * 16, 16)]
                b_ti[sl] = b_t[pl.ds(k0 + g * 16, 16)]
            cph = pltpu.async_copy(x_hbm.at[b_hi], b_xh, sem)
            cpt = pltpu.async_copy(x_hbm.at[b_ti], b_xt, sem)
            cpr = pltpu.async_copy(rel_hbm.at[b_ri], b_rr, sem)
            cph.wait()
            cpt.wait()
            cpr.wait()

            def ftrip(rr, _):
                acc = jnp.zeros((16,), jnp.float32)

                def fdim(g, a):
                    sl = pl.ds(g * 16, 16)
                    return a + b_xh[rr, sl] * b_rr[rr, sl] * b_xt[rr, sl]

                acc = lax.fori_loop(0, D // 16, fdim, acc)
                tot = _hsum(acc)
                lane = rr - (rr // 16) * 16
                q0 = k0 + (rr // 16) * 16
                old = b_res[pl.ds(q0, 16)]
                b_res[pl.ds(q0, 16)] = jnp.where(iota16 == lane, tot, old)
                return 0

            lax.fori_loop(0, KT, ftrip, 0)

        pltpu.sync_copy(b_res, out_hbm.at[pl.ds(base, TPT)])

    return body(x, relt, h, r, t)


def kernel(edge_index, edge_type, h, r, t, emb, comp1, bases1, root1, bias1,
           comp2, bases2, root2, bias2, rel):
    npad = NE_PAD - N_EDGES
    src = jnp.concatenate(
        [edge_index[0], jnp.zeros((npad,), jnp.int32)])
    dst = jnp.concatenate(
        [edge_index[1], jnp.full((npad,), DUMMY, jnp.int32)])
    typ = jnp.concatenate(
        [edge_type, jnp.zeros((npad,), jnp.int32)])

    c0, c1 = _counts(dst, typ)
    wtab = _winv(
        c0.reshape(640, 128), c1.reshape(640, 128)
    ).reshape(CNT_PAD)

    wcat1 = _wcat(comp1, bases1, root1)
    wcat2 = _wcat(comp2, bases2, root2)
    zpad = jnp.zeros((1, N_REL * D), jnp.float32)
    bcat1 = jnp.concatenate([bias1.reshape(1, D), zpad], axis=1)
    bcat2 = jnp.concatenate([bias2.reshape(1, D), zpad], axis=1)

    y1 = _mm1(emb, wcat1, bcat1)
    a1_0a, a1_0b, a1_1a, a1_1b = _aggregate(
        y1.reshape(N_NODES * NPLANE * 2, DH), src, dst, typ, wtab)
    a1_0 = jnp.concatenate([a1_0a, a1_0b], axis=1)
    a1_1 = jnp.concatenate([a1_1a, a1_1b], axis=1)
    y2 = _mm2(y1[:, 0:D], a1_0, a1_1, wcat2, bcat2)
    a2_0a, a2_0b, a2_1a, a2_1b = _aggregate(
        y2.reshape(N_NODES * NPLANE * 2, DH), src, dst, typ, wtab)
    a2_0 = jnp.concatenate([a2_0a, a2_0b], axis=1)
    a2_1 = jnp.concatenate([a2_1a, a2_1b], axis=1)
    xf = _finalx(y2[:, 0:D], a2_0, a2_1)
    return _decoder(xf, rel, h, r, t)
